# Initial kernel scaffold; baseline (speedup 1.0000x reference)
import functools

import jax
import jax.numpy as jnp
from jax import lax
from jax.experimental import pallas as pl
from jax.experimental.pallas import tpu as pltpu
from jax.experimental.pallas import tpu_sc as plsc

N = 10000
NP = 10240
E = 320000
K = 128
CH = 79
EPT = K * CH
NT = 32
EP = EPT * NT
D_IN, D_HID, D_OUT = 128, 64, 128
RPT = NP // 16

_mesh = plsc.VectorSubcoreMesh(core_axis_name="c", subcore_axis_name="s")


@functools.partial(
    pl.kernel,
    mesh=_mesh,
    out_type=jax.ShapeDtypeStruct((2, NP, 16), jnp.float32),
    scratch_types=[
        pltpu.VMEM((CH, K), jnp.int32),
        pltpu.VMEM((K, 16), jnp.float32),
        pltpu.VMEM((K, 16), jnp.float32),
        pltpu.VMEM_SHARED((NP, 16), jnp.float32),
    ],
)
def _deg(dst_hbm, out_hbm, dst_v, ones_v, zb_v, acc):
    c = lax.axis_index("c")
    s = lax.axis_index("s")
    tl = c * 16 + s
    pltpu.sync_copy(dst_hbm.at[tl], dst_v)

    @pl.loop(0, K)
    def _fill(i):
        ones_v[i, :] = jnp.ones((16,), jnp.float32)
        zb_v[i, :] = jnp.zeros((16,), jnp.float32)

    @pl.loop(0, RPT // K)
    def _zero(k):
        pltpu.sync_copy(zb_v, acc.at[pl.ds(s * RPT + k * K, K)])

    plsc.subcore_barrier()

    @pl.loop(0, CH)
    def _hist(j):
        pltpu.sync_copy(ones_v, acc.at[dst_v.at[j]], add=True)

    plsc.subcore_barrier()

    @pl.loop(0, RPT // K)
    def _out(k):
        start = s * RPT + k * K
        pltpu.sync_copy(acc.at[pl.ds(start, K)], out_hbm.at[c, pl.ds(start, K)])


@functools.partial(
    pl.kernel,
    mesh=_mesh,
    out_type=jax.ShapeDtypeStruct((2, NP, D_HID), jnp.float32),
    scratch_types=[
        pltpu.VMEM((CH, K), jnp.int32),
        pltpu.VMEM((CH, K), jnp.int32),
        pltpu.VMEM((K, D_HID), jnp.float32),
        pltpu.VMEM((K, D_HID), jnp.float32),
        pltpu.VMEM_SHARED((NP, D_HID), jnp.float32),
        pltpu.SemaphoreType.DMA,
    ],
)
def _agg(g_hbm, src_hbm, dst_hbm, out_hbm, src_v, dst_v, rows_v, zb_v, acc, sem):
    c = lax.axis_index("c")
    s = lax.axis_index("s")
    tl = c * 16 + s
    pltpu.sync_copy(src_hbm.at[tl], src_v)
    pltpu.sync_copy(dst_hbm.at[tl], dst_v)

    @pl.loop(0, K)
    def _fill(i):
        @pl.loop(0, D_HID, step=16)
        def _fill16(j):
            zb_v[i, pl.ds(j, 16)] = jnp.zeros((16,), jnp.float32)

    @pl.loop(0, RPT // K)
    def _zero(k):
        pltpu.sync_copy(zb_v, acc.at[pl.ds(s * RPT + k * K, K)])

    plsc.subcore_barrier()

    @pl.loop(0, CH)
    def _edges(j):
        pltpu.async_copy(g_hbm.at[src_v.at[j]], rows_v, sem).wait()
        pltpu.sync_copy(rows_v, acc.at[dst_v.at[j]], add=True)

    plsc.subcore_barrier()

    @pl.loop(0, RPT // K)
    def _out(k):
        start = s * RPT + k * K
        pltpu.sync_copy(acc.at[pl.ds(start, K)], out_hbm.at[c, pl.ds(start, K)])


def _mm1_body(x_ref, w_ref, o_ref):
    o_ref[...] = lax.dot(x_ref[...], w_ref[...],
                         precision=lax.Precision.HIGHEST)


_mm1 = pl.pallas_call(
    _mm1_body,
    out_shape=jax.ShapeDtypeStruct((NP, D_HID), jnp.float32),
)


def _dinv_of(deg_ref):
    cnt = deg_ref[0, :, 0:1] + deg_ref[1, :, 0:1]
    return lax.rsqrt(cnt + 1.0)


def _scale_body(deg_ref, h_ref, o_ref):
    o_ref[...] = _dinv_of(deg_ref) * h_ref[...]


_scale = pl.pallas_call(
    _scale_body,
    out_shape=jax.ShapeDtypeStruct((NP, D_HID), jnp.float32),
)


def _mid_body(p_ref, g_ref, deg_ref, b_ref, o_ref):
    dinv = _dinv_of(deg_ref)
    a = dinv * (p_ref[0] + p_ref[1] + g_ref[...]) + b_ref[...]
    o_ref[...] = dinv * jnp.maximum(a, 0.0)


_mid = pl.pallas_call(
    _mid_body,
    out_shape=jax.ShapeDtypeStruct((NP, D_HID), jnp.float32),
)


def _final_body(p_ref, g_ref, deg_ref, w_ref, b_ref, o_ref):
    dinv = _dinv_of(deg_ref)
    a = dinv * (p_ref[0] + p_ref[1] + g_ref[...])
    o_ref[...] = lax.dot(a, w_ref[...],
                         precision=lax.Precision.HIGHEST) + b_ref[...]


_final = pl.pallas_call(
    _final_body,
    out_shape=jax.ShapeDtypeStruct((NP, D_OUT), jnp.float32),
)


@jax.jit
def kernel(x, edge_index, W1, b1, W2, b2):
    src = edge_index[0]
    dst = edge_index[1]
    pad_e = EP - E
    src_p = jnp.concatenate(
        [src, jnp.zeros((pad_e,), jnp.int32)]).reshape(NT, CH, K)
    dst_p = jnp.concatenate(
        [dst, jnp.full((pad_e,), N, jnp.int32)]).reshape(NT, CH, K)
    x_p = jnp.pad(x, ((0, NP - N), (0, 0)))

    deg = _deg(dst_p)
    h1 = _mm1(x_p, W1)
    g1 = _scale(deg, h1)
    p1 = _agg(g1, src_p, dst_p)
    g2 = _mid(p1, g1, deg, b1.reshape(1, D_HID))
    p2 = _agg(g2, src_p, dst_p)
    out = _final(p2, g2, deg, W2, b2.reshape(1, D_OUT))
    return out[:N]


# speedup vs baseline: 20.0094x; 20.0094x over previous
import functools

import jax
import jax.numpy as jnp
from jax import lax
from jax.experimental import pallas as pl
from jax.experimental.pallas import tpu as pltpu
from jax.experimental.pallas import tpu_sc as plsc

N = 10000
NP = 10240
E = 320000
K = 128
CH = 79
EPT = K * CH
NT = 32
EP = EPT * NT
D_IN, D_HID, D_OUT = 128, 64, 128
RPT = NP // 16

_mesh = plsc.VectorSubcoreMesh(core_axis_name="c", subcore_axis_name="s")
_sc_params = pltpu.CompilerParams(use_tc_tiling_on_sc=False)


@functools.partial(
    pl.kernel,
    mesh=_mesh,
    out_type=jax.ShapeDtypeStruct((2, NP, 16), jnp.float32),
    scratch_types=[
        pltpu.VMEM((CH, K), jnp.int32),
        pltpu.VMEM((K, 16), jnp.float32),
        pltpu.VMEM((K, 16), jnp.float32),
        pltpu.VMEM_SHARED((NP, 16), jnp.float32),
    ],
    compiler_params=_sc_params,
)
def _deg(dst_hbm, out_hbm, dst_v, ones_v, zb_v, acc):
    c = lax.axis_index("c")
    s = lax.axis_index("s")
    tl = c * 16 + s
    pltpu.sync_copy(dst_hbm.at[tl], dst_v)

    @pl.loop(0, K)
    def _fill(i):
        ones_v[i, :] = jnp.ones((16,), jnp.float32)
        zb_v[i, :] = jnp.zeros((16,), jnp.float32)

    @pl.loop(0, RPT // K)
    def _zero(k):
        pltpu.sync_copy(zb_v, acc.at[pl.ds(s * RPT + k * K, K)])

    plsc.subcore_barrier()

    @pl.loop(0, CH)
    def _hist(j):
        pltpu.sync_copy(ones_v, acc.at[dst_v.at[j]], add=True)

    plsc.subcore_barrier()

    @pl.loop(0, RPT // K)
    def _out(k):
        start = s * RPT + k * K
        pltpu.sync_copy(acc.at[pl.ds(start, K)], out_hbm.at[c, pl.ds(start, K)])


@functools.partial(
    pl.kernel,
    mesh=_mesh,
    out_type=jax.ShapeDtypeStruct((2, NP, D_HID), jnp.float32),
    scratch_types=[
        pltpu.VMEM((CH, K), jnp.int32),
        pltpu.VMEM((CH, K), jnp.int32),
        pltpu.VMEM((K, D_HID), jnp.float32),
        pltpu.VMEM((K, D_HID), jnp.float32),
        pltpu.VMEM_SHARED((NP, D_HID), jnp.float32),
        pltpu.SemaphoreType.DMA,
    ],
    compiler_params=_sc_params,
)
def _agg(g_hbm, src_hbm, dst_hbm, out_hbm, src_v, dst_v, rows_v, zb_v, acc, sem):
    c = lax.axis_index("c")
    s = lax.axis_index("s")
    tl = c * 16 + s
    pltpu.sync_copy(src_hbm.at[tl], src_v)
    pltpu.sync_copy(dst_hbm.at[tl], dst_v)

    @pl.loop(0, K)
    def _fill(i):
        @pl.loop(0, D_HID, step=16)
        def _fill16(j):
            zb_v[i, pl.ds(j, 16)] = jnp.zeros((16,), jnp.float32)

    @pl.loop(0, RPT // K)
    def _zero(k):
        pltpu.sync_copy(zb_v, acc.at[pl.ds(s * RPT + k * K, K)])

    plsc.subcore_barrier()

    @pl.loop(0, CH)
    def _edges(j):
        pltpu.async_copy(g_hbm.at[src_v.at[j]], rows_v, sem).wait()
        pltpu.sync_copy(rows_v, acc.at[dst_v.at[j]], add=True)

    plsc.subcore_barrier()

    @pl.loop(0, RPT // K)
    def _out(k):
        start = s * RPT + k * K
        pltpu.sync_copy(acc.at[pl.ds(start, K)], out_hbm.at[c, pl.ds(start, K)])


def _mm1_body(x_ref, w_ref, o_ref):
    o_ref[...] = lax.dot(x_ref[...], w_ref[...],
                         precision=lax.Precision.HIGHEST)


_mm1 = pl.pallas_call(
    _mm1_body,
    out_shape=jax.ShapeDtypeStruct((NP, D_HID), jnp.float32),
)


def _dinv_of(deg_ref):
    cnt = deg_ref[0, :, 0:1] + deg_ref[1, :, 0:1]
    return lax.rsqrt(cnt + 1.0)


def _scale_body(deg_ref, h_ref, o_ref):
    o_ref[...] = _dinv_of(deg_ref) * h_ref[...]


_scale = pl.pallas_call(
    _scale_body,
    out_shape=jax.ShapeDtypeStruct((NP, D_HID), jnp.float32),
)


def _mid_body(p_ref, g_ref, deg_ref, b_ref, o_ref):
    dinv = _dinv_of(deg_ref)
    a = dinv * (p_ref[0] + p_ref[1] + g_ref[...]) + b_ref[...]
    o_ref[...] = dinv * jnp.maximum(a, 0.0)


_mid = pl.pallas_call(
    _mid_body,
    out_shape=jax.ShapeDtypeStruct((NP, D_HID), jnp.float32),
)


def _final_body(p_ref, g_ref, deg_ref, w_ref, b_ref, o_ref):
    dinv = _dinv_of(deg_ref)
    a = dinv * (p_ref[0] + p_ref[1] + g_ref[...])
    o_ref[...] = lax.dot(a, w_ref[...],
                         precision=lax.Precision.HIGHEST) + b_ref[...]


_final = pl.pallas_call(
    _final_body,
    out_shape=jax.ShapeDtypeStruct((NP, D_OUT), jnp.float32),
)


@jax.jit
def kernel(x, edge_index, W1, b1, W2, b2):
    src = edge_index[0]
    dst = edge_index[1]
    pad_e = EP - E
    src_p = jnp.concatenate(
        [src, jnp.zeros((pad_e,), jnp.int32)]).reshape(NT, CH, K)
    dst_p = jnp.concatenate(
        [dst, jnp.full((pad_e,), N, jnp.int32)]).reshape(NT, CH, K)
    x_p = jnp.pad(x, ((0, NP - N), (0, 0)))

    deg = _deg(dst_p)
    h1 = _mm1(x_p, W1)
    g1 = _scale(deg, h1)
    p1 = _agg(g1, src_p, dst_p)
    g2 = _mid(p1, g1, deg, b1.reshape(1, D_HID))
    p2 = _agg(g2, src_p, dst_p)
    out = _final(p2, g2, deg, W2, b2.reshape(1, D_OUT))
    return out[:N]
